# baseline (speedup 1.0000x reference)
import jax
import jax.numpy as jnp
from jax import lax
from jax.experimental import pallas as pl
from jax.experimental.pallas import tpu as pltpu

N = 1_000_000
LAM = 0.1
BS = 1048576
NB = -(-N // BS)
ROWS = BS // 128


def _body(sal_ref, pi_ref, di_ref, o_ref, w_v, s_v):
    i = pl.program_id(0)

    @pl.when(i < NB)
    def _():
        sal = sal_ref[...].reshape(ROWS, 128)
        expo = (pi_ref[...] + di_ref[...]).reshape(ROWS, 128)
        w = jnp.maximum(sal, 1e-8) * jnp.exp(expo * -LAM)
        w_v[pl.ds(i * BS, BS)] = w.reshape(BS)

        @pl.when(i < NB - 1)
        def _():
            s_v[0] = jnp.where(i == 0, 0.0, s_v[0]) + jnp.sum(w)

        @pl.when(i == NB - 1)
        def _():
            idx = (
                i * BS
                + lax.broadcasted_iota(jnp.int32, (ROWS, 128), 0) * 128
                + lax.broadcasted_iota(jnp.int32, (ROWS, 128), 1)
            )
            s_v[0] = s_v[0] + jnp.sum(jnp.where(idx < N, w, 0.0))

    @pl.when(i >= NB)
    def _():
        inv = 1.0 / jnp.maximum(s_v[0], 1e-8)
        o_ref[...] = w_v[pl.ds((i - NB) * BS, BS)] * inv


_call = pl.pallas_call(
    _body,
    grid=(2 * NB,),
    in_specs=[
        pl.BlockSpec((BS,), lambda i: (jnp.minimum(i, NB - 1),)),
        pl.BlockSpec((BS,), lambda i: (jnp.minimum(i, NB - 1),)),
        pl.BlockSpec((BS,), lambda i: (jnp.minimum(i, NB - 1),)),
    ],
    out_specs=pl.BlockSpec((BS,), lambda i: (jnp.maximum(i - NB, 0),)),
    out_shape=jax.ShapeDtypeStruct((N,), jnp.float32),
    scratch_shapes=[
        pltpu.VMEM((NB * BS,), jnp.float32),
        pltpu.SMEM((1,), jnp.float32),
    ],
)


def kernel(saliences, Pi_q, delta_identity):
    return _call(saliences, Pi_q, delta_identity)


# speedup vs baseline: 1.1557x; 1.1557x over previous
import jax
import jax.numpy as jnp
from jax import lax
from jax.experimental import pallas as pl
from jax.experimental.pallas import tpu as pltpu

N = 1_000_000
LAM = 0.1
BS = 524288
NB = -(-N // BS)
ROWS = BS // 128


def _body(sal_ref, pi_ref, di_ref, o_ref, w_v, s_v):
    i = pl.program_id(0)

    @pl.when(i < NB)
    def _():
        sal = sal_ref[...].reshape(ROWS, 128)
        expo = (pi_ref[...] + di_ref[...]).reshape(ROWS, 128)
        w = jnp.maximum(sal, 1e-8) * jnp.exp(expo * -LAM)
        w_v[pl.ds(i * BS, BS)] = w.reshape(BS)

        @pl.when(i < NB - 1)
        def _():
            s_v[0] = jnp.where(i == 0, 0.0, s_v[0]) + jnp.sum(w)

        @pl.when(i == NB - 1)
        def _():
            idx = (
                i * BS
                + lax.broadcasted_iota(jnp.int32, (ROWS, 128), 0) * 128
                + lax.broadcasted_iota(jnp.int32, (ROWS, 128), 1)
            )
            s_v[0] = s_v[0] + jnp.sum(jnp.where(idx < N, w, 0.0))

    @pl.when(i >= NB)
    def _():
        inv = 1.0 / jnp.maximum(s_v[0], 1e-8)
        o_ref[...] = w_v[pl.ds((i - NB) * BS, BS)] * inv


_call = pl.pallas_call(
    _body,
    grid=(2 * NB,),
    in_specs=[
        pl.BlockSpec((BS,), lambda i: (jnp.minimum(i, NB - 1),)),
        pl.BlockSpec((BS,), lambda i: (jnp.minimum(i, NB - 1),)),
        pl.BlockSpec((BS,), lambda i: (jnp.minimum(i, NB - 1),)),
    ],
    out_specs=pl.BlockSpec((BS,), lambda i: (jnp.maximum(i - NB, 0),)),
    out_shape=jax.ShapeDtypeStruct((N,), jnp.float32),
    scratch_shapes=[
        pltpu.VMEM((NB * BS,), jnp.float32),
        pltpu.SMEM((1,), jnp.float32),
    ],
)


def kernel(saliences, Pi_q, delta_identity):
    return _call(saliences, Pi_q, delta_identity)
